# baseline (speedup 1.0000x reference)
import dataclasses
import functools

import jax
import jax.numpy as jnp
from jax import lax
from jax.experimental import pallas as pl
from jax.experimental.pallas import tpu as pltpu
from jax.experimental.pallas import tpu_sc as plsc

N1, F0 = 400000, 4
N2, F1 = 100000, 8
N3, F2 = 20000, 5
N4, F3 = 5000, 4
N1P = 409600
N2P = 102400
N3P = 20480
N4P = 5120

NUM_WORKERS = 32
LANES = 16
UNROLL = 2


def _mesh():
    return plsc.VectorSubcoreMesh(core_axis_name="c", subcore_axis_name="s")


def _compiler_params():
    cp = pltpu.CompilerParams()
    if "needs_layout_passes" in pltpu.CompilerParams.__dataclass_fields__:
        cp = dataclasses.replace(cp, needs_layout_passes=False)
    return cp


def _wid():
    return lax.axis_index("s") * 2 + lax.axis_index("c")


def _reduce_chunk(idx_v, g_ref, out_v, ch, fan, is_prod, decode_literals,
                  out_off=0):
    iota_f = lax.iota(jnp.int32, LANES) * fan

    @pl.loop(0, ch, step=LANES * UNROLL)
    def _(i):
        for u in range(UNROLL):
            iu = i + u * LANES
            acc = None
            for j in range(fan):
                pos = iota_f + (iu * fan + j)
                raw = plsc.load_gather(idx_v, [pos])
                if decode_literals:
                    var = (raw >> 1) - 1
                    g = plsc.load_gather(g_ref, [var])
                    sf = (raw & 1).astype(jnp.float32)
                    g = jnp.abs(g - sf)
                else:
                    g = plsc.load_gather(g_ref, [raw])
                if acc is None:
                    acc = g
                else:
                    acc = acc * g if is_prod else acc + g
            out_v[pl.ds(out_off + iu, LANES)] = acc


def _reduce_positional(g_v, out_v, ch, fan, is_prod):
    iota_f = lax.iota(jnp.int32, LANES) * fan

    @pl.loop(0, ch, step=LANES * UNROLL)
    def _(i):
        for u in range(UNROLL):
            iu = i + u * LANES
            acc = None
            for j in range(fan):
                pos = iota_f + (iu * fan + j)
                g = plsc.load_gather(g_v, [pos])
                if acc is None:
                    acc = g
                else:
                    acc = acc * g if is_prod else acc + g
            out_v[pl.ds(iu, LANES)] = acc


XS_OFF = 6
XS_LEN = 200016
NB_VARS = 100000
ENC_PER_SUB = 6256


def _layer0(x_pos, idx):
    per_w = N1P // NUM_WORKERS
    ch = 800
    sp = 320
    cp_ = ch - sp
    fan = F0
    nsub_full = per_w // ch
    nsub_tail = (N1 - (NUM_WORKERS - 1) * per_w) // ch

    @functools.partial(
        pl.kernel,
        out_type=jax.ShapeDtypeStruct((N1P,), jnp.float32),
        mesh=_mesh(),
        compiler_params=_compiler_params(),
        scratch_types=[
            pltpu.VMEM_SHARED((XS_LEN,), jnp.float32),
            pltpu.VMEM((N2,), jnp.float32),
            pltpu.VMEM((sp * fan,), jnp.int32),
            pltpu.VMEM((sp * fan,), jnp.int32),
            pltpu.VMEM((cp_ * fan,), jnp.int32),
            pltpu.VMEM((cp_ * fan,), jnp.int32),
            pltpu.VMEM((sp * fan,), jnp.float32),
            pltpu.VMEM((ch,), jnp.float32),
            pltpu.VMEM((ch,), jnp.float32),
            pltpu.VMEM((784,), jnp.float32),
            pltpu.VMEM((1568,), jnp.float32),
            pltpu.SemaphoreType.DMA,
            pltpu.SemaphoreType.DMA,
            pltpu.SemaphoreType.DMA,
            pltpu.SemaphoreType.DMA,
            pltpu.SemaphoreType.DMA,
            pltpu.SemaphoreType.DMA,
        ],
    )
    def k(tab_hbm, idx_hbm, out_hbm, x_s, tab_v, isa, isb_, ica, icb_, gq,
          oa, ob_, pos_b, sc_b, tsem, isem_a, isem_b, gsem, osem_a, osem_b):
        w = _wid()
        sid = lax.axis_index("s")
        base = w * per_w
        isb = [isa, isb_]
        icb = [ica, icb_]
        ob = [oa, ob_]
        isems = [isem_a, isem_b]
        osems = [osem_a, osem_b]
        iota2 = lax.iota(jnp.int32, LANES) * 2
        tab_cp = pltpu.async_copy(tab_hbm, tab_v, tsem)

        def encode_round(v0, nv, dst_off):
            pltpu.sync_copy(tab_hbm.at[pl.ds(v0, nv)], pos_b.at[pl.ds(0, nv)])

            @pl.loop(0, nv, step=LANES)
            def _(i):
                pos = pos_b[pl.ds(i, LANES)]
                te = iota2 + 2 * i
                plsc.store_scatter(sc_b, [te], pos)
                plsc.store_scatter(sc_b, [te + 1], 1.0 - pos)

            pltpu.sync_copy(sc_b.at[pl.ds(0, 2 * nv)],
                            x_s.at[pl.ds(dst_off, 2 * nv)])

        vb = sid * ENC_PER_SUB
        db = 8 + sid * (2 * ENC_PER_SUB)

        @pl.when(sid < LANES - 1)
        def _():
            for r, nv in enumerate([784] * 7 + [768]):
                encode_round(vb + 784 * r, nv, db + 1568 * r)

        @pl.when(sid == LANES - 1)
        def _():
            for r, nv in enumerate([784] * 7 + [672]):
                encode_round(vb + 784 * r, nv, db + 1568 * r)

        plsc.subcore_barrier()

        def span(nsub):
            iscp = [None] * nsub
            iccp = [None] * nsub
            ocp = [None] * nsub

            def fire_idx(t):
                e0 = (base + t * ch) * fan
                iscp[t] = pltpu.async_copy(
                    idx_hbm.at[pl.ds(e0, sp * fan)], isb[t % 2],
                    isems[t % 2])
                iccp[t] = pltpu.async_copy(
                    idx_hbm.at[pl.ds(e0 + sp * fan, cp_ * fan)], icb[t % 2],
                    isems[t % 2])

            fire_idx(0)
            if nsub > 1:
                fire_idx(1)
            for t in range(nsub):
                iscp[t].wait()

                @pl.loop(0, sp * fan, step=LANES)
                def _(i):
                    sl = pl.ds(i, LANES)
                    isb[t % 2][sl] = isb[t % 2][sl] + XS_OFF

                gcp = pltpu.async_copy(x_s.at[isb[t % 2]], gq, gsem)
                iccp[t].wait()
                if t == 0:
                    tab_cp.wait()
                if t >= 2:
                    ocp[t - 2].wait()
                _reduce_chunk(icb[t % 2], tab_v, ob[t % 2], cp_, fan, True,
                              True, out_off=sp)
                gcp.wait()
                if t + 2 < nsub:
                    fire_idx(t + 2)
                _reduce_positional(gq, ob[t % 2], sp, fan, True)
                ocp[t] = pltpu.async_copy(
                    ob[t % 2], out_hbm.at[pl.ds(base + t * ch, ch)],
                    osems[t % 2])
            for t in range(max(nsub - 2, 0), nsub):
                ocp[t].wait()

        @pl.when(w < NUM_WORKERS - 1)
        def _():
            span(nsub_full)

        @pl.when(w == NUM_WORKERS - 1)
        def _():
            span(nsub_tail)

    return k(x_pos, idx)


def _layer1(table, idx):
    fan = F1
    per_w = N2P // NUM_WORKERS
    nsub_full = 4
    ch = per_w // nsub_full
    cw = ch * fan
    slice_w = N1P // LANES
    nsub_tail = (N2 - (NUM_WORKERS - 1) * per_w) // ch

    @functools.partial(
        pl.kernel,
        out_type=jax.ShapeDtypeStruct((N2P,), jnp.float32),
        mesh=_mesh(),
        compiler_params=_compiler_params(),
        scratch_types=[
            pltpu.VMEM_SHARED((N1P,), jnp.float32),
            pltpu.VMEM((cw,), jnp.int32),
            pltpu.VMEM((cw,), jnp.int32),
            pltpu.VMEM((cw,), jnp.float32),
            pltpu.VMEM((cw,), jnp.float32),
            pltpu.VMEM((ch,), jnp.float32),
            pltpu.VMEM((ch,), jnp.float32),
            pltpu.SemaphoreType.DMA,
            pltpu.SemaphoreType.DMA,
            pltpu.SemaphoreType.DMA,
            pltpu.SemaphoreType.DMA,
            pltpu.SemaphoreType.DMA,
            pltpu.SemaphoreType.DMA,
        ],
    )
    def k(tab_hbm, idx_hbm, out_hbm, tab_s, ia, ib_, ga, gb_, oa, ob_,
          isem_a, isem_b, gsem_a, gsem_b, osem_a, osem_b):
        sid = lax.axis_index("s")
        pltpu.sync_copy(tab_hbm.at[pl.ds(sid * slice_w, slice_w)],
                        tab_s.at[pl.ds(sid * slice_w, slice_w)])
        plsc.subcore_barrier()
        w = _wid()
        base = w * per_w
        ib = [ia, ib_]
        gb = [ga, gb_]
        ob = [oa, ob_]
        isems = [isem_a, isem_b]
        gsems = [gsem_a, gsem_b]
        osems = [osem_a, osem_b]

        def span(nsub):
            icp = [None] * nsub
            gcp = [None] * nsub
            ocp = [None] * nsub

            def fire_idx(t):
                return pltpu.async_copy(
                    idx_hbm.at[pl.ds(base * fan + t * cw, cw)],
                    ib[t % 2], isems[t % 2])

            def fire_gather(t):
                src = tab_s if t % 2 == 0 else tab_hbm
                return pltpu.async_copy(src.at[ib[t % 2]], gb[t % 2],
                                        gsems[t % 2])

            icp[0] = fire_idx(0)
            if nsub > 1:
                icp[1] = fire_idx(1)
            icp[0].wait()
            gcp[0] = fire_gather(0)
            for t in range(nsub):
                if t + 1 < nsub:
                    icp[t + 1].wait()
                    gcp[t + 1] = fire_gather(t + 1)
                gcp[t].wait()
                if t + 2 < nsub:
                    icp[t + 2] = fire_idx(t + 2)
                if t >= 2:
                    ocp[t - 2].wait()
                _reduce_positional(gb[t % 2], ob[t % 2], ch, fan, False)
                ocp[t] = pltpu.async_copy(
                    ob[t % 2], out_hbm.at[pl.ds(base + t * ch, ch)],
                    osems[t % 2])
            for t in range(max(nsub - 2, 0), nsub):
                ocp[t].wait()

        @pl.when(w < NUM_WORKERS - 1)
        def _():
            span(nsub_full)

        @pl.when(w == NUM_WORKERS - 1)
        def _():
            span(nsub_tail)

    return k(table, idx)


def _layers23(table, idx2, idx3):
    ch2 = N3P // LANES
    ch3 = N4P // NUM_WORKERS
    slice_w = N2P // LANES

    @functools.partial(
        pl.kernel,
        out_type=jax.ShapeDtypeStruct((N4P,), jnp.float32),
        mesh=_mesh(),
        compiler_params=_compiler_params(),
        scratch_types=[
            pltpu.VMEM_SHARED((N2P,), jnp.float32),
            pltpu.VMEM_SHARED((N3P,), jnp.float32),
            pltpu.VMEM((ch2 * F2,), jnp.int32),
            pltpu.VMEM((ch2 * F2,), jnp.float32),
            pltpu.VMEM((ch2,), jnp.float32),
            pltpu.VMEM((ch3 * F3,), jnp.int32),
            pltpu.VMEM((ch3 * F3,), jnp.float32),
            pltpu.VMEM((ch3,), jnp.float32),
            pltpu.SemaphoreType.DMA,
            pltpu.SemaphoreType.DMA,
        ],
    )
    def k(tab_hbm, idx2_hbm, idx3_hbm, out_hbm, l1_s, l2_s, i2, g2, o2,
          i3, g3, o3, sem_a, sem_b):
        sid = lax.axis_index("s")
        core = lax.axis_index("c")
        base3 = core * (N4P // 2) + sid * ch3
        pltpu.sync_copy(tab_hbm.at[pl.ds(sid * slice_w, slice_w)],
                        l1_s.at[pl.ds(sid * slice_w, slice_w)])
        icp2 = pltpu.async_copy(
            idx2_hbm.at[pl.ds(sid * ch2 * F2, ch2 * F2)], i2, sem_a)
        icp3 = pltpu.async_copy(
            idx3_hbm.at[pl.ds(base3 * F3, ch3 * F3)], i3, sem_b)
        plsc.subcore_barrier()
        icp2.wait()
        pltpu.async_copy(l1_s.at[i2], g2, sem_a).wait()
        _reduce_positional(g2, o2, ch2, F2, True)
        pltpu.sync_copy(o2, l2_s.at[pl.ds(sid * ch2, ch2)])
        plsc.subcore_barrier()
        icp3.wait()
        pltpu.async_copy(l2_s.at[i3], g3, sem_b).wait()
        _reduce_positional(g3, o3, ch3, F3, False)
        pltpu.sync_copy(o3, out_hbm.at[pl.ds(base3, ch3)])

    return k(table, idx2, idx3)


def kernel(x_pos, ix_in0, ix_out0, ix_in1, ix_out1, ix_in2, ix_out2,
           ix_in3, ix_out3):
    del ix_out0, ix_out1, ix_out2, ix_out3

    ix2 = jnp.pad(ix_in2, (0, (N3P - N3) * F2))
    ix3 = jnp.pad(ix_in3, (0, (N4P - N4) * F3))

    l0 = _layer0(x_pos, ix_in0)
    l1 = _layer1(l0, ix_in1)
    l3 = _layers23(l1, ix2, ix3)
    return l3[:N4]


# speedup vs baseline: 1.1726x; 1.1726x over previous
import dataclasses
import functools

import jax
import jax.numpy as jnp
from jax import lax
from jax.experimental import pallas as pl
from jax.experimental.pallas import tpu as pltpu
from jax.experimental.pallas import tpu_sc as plsc

N1, F0 = 400000, 4
N2, F1 = 100000, 8
N3, F2 = 20000, 5
N4, F3 = 5000, 4
N1P = 409600
N2P = 102400
N3P = 20480
N4P = 5120

NUM_WORKERS = 32
LANES = 16
UNROLL = 2


def _mesh():
    return plsc.VectorSubcoreMesh(core_axis_name="c", subcore_axis_name="s")


def _compiler_params():
    cp = pltpu.CompilerParams()
    if "needs_layout_passes" in pltpu.CompilerParams.__dataclass_fields__:
        cp = dataclasses.replace(cp, needs_layout_passes=False)
    return cp


def _wid():
    return lax.axis_index("s") * 2 + lax.axis_index("c")


def _reduce_chunk(idx_v, g_ref, out_v, ch, fan, is_prod, decode_literals):
    iota_f = lax.iota(jnp.int32, LANES) * fan

    @pl.loop(0, ch, step=LANES * UNROLL)
    def _(i):
        for u in range(UNROLL):
            iu = i + u * LANES
            acc = None
            for j in range(fan):
                pos = iota_f + (iu * fan + j)
                raw = plsc.load_gather(idx_v, [pos])
                if decode_literals:
                    var = (raw >> 1) - 1
                    g = plsc.load_gather(g_ref, [var])
                    sf = (raw & 1).astype(jnp.float32)
                    g = jnp.abs(g - sf)
                else:
                    g = plsc.load_gather(g_ref, [raw])
                if acc is None:
                    acc = g
                else:
                    acc = acc * g if is_prod else acc + g
            out_v[pl.ds(iu, LANES)] = acc


def _reduce_positional(g_v, out_v, ch, fan, is_prod):
    iota_f = lax.iota(jnp.int32, LANES) * fan

    @pl.loop(0, ch, step=LANES * UNROLL)
    def _(i):
        for u in range(UNROLL):
            iu = i + u * LANES
            acc = None
            for j in range(fan):
                pos = iota_f + (iu * fan + j)
                g = plsc.load_gather(g_v, [pos])
                if acc is None:
                    acc = g
                else:
                    acc = acc * g if is_prod else acc + g
            out_v[pl.ds(iu, LANES)] = acc


def _layer0(x_pos, idx):
    per_w = N1P // NUM_WORKERS
    ch = 1600
    fan = F0
    nsub_full = per_w // ch
    nsub_tail = (N1 - (NUM_WORKERS - 1) * per_w) // ch

    @functools.partial(
        pl.kernel,
        out_type=jax.ShapeDtypeStruct((N1P,), jnp.float32),
        mesh=_mesh(),
        compiler_params=_compiler_params(),
        scratch_types=[
            pltpu.VMEM((N2,), jnp.float32),
            pltpu.VMEM((ch * fan,), jnp.int32),
            pltpu.VMEM((ch * fan,), jnp.int32),
            pltpu.VMEM((ch,), jnp.float32),
            pltpu.VMEM((ch,), jnp.float32),
            pltpu.SemaphoreType.DMA,
            pltpu.SemaphoreType.DMA,
            pltpu.SemaphoreType.DMA,
            pltpu.SemaphoreType.DMA,
            pltpu.SemaphoreType.DMA,
        ],
    )
    def k(tab_hbm, idx_hbm, out_hbm, tab_v, ia, ib_, oa, ob_, tsem,
          isem_a, isem_b, osem_a, osem_b):
        w = _wid()
        base = w * per_w
        ib = [ia, ib_]
        ob = [oa, ob_]
        isems = [isem_a, isem_b]
        osems = [osem_a, osem_b]
        tab_cp = pltpu.async_copy(tab_hbm, tab_v, tsem)

        def span(nsub):
            icp = [None] * nsub
            ocp = [None] * nsub

            def fire_idx(t):
                return pltpu.async_copy(
                    idx_hbm.at[pl.ds((base + t * ch) * fan, ch * fan)],
                    ib[t % 2], isems[t % 2])

            icp[0] = fire_idx(0)
            if nsub > 1:
                icp[1] = fire_idx(1)
            for t in range(nsub):
                if t >= 1 and t + 1 < nsub:
                    icp[t + 1] = fire_idx(t + 1)
                icp[t].wait()
                if t == 0:
                    tab_cp.wait()
                if t >= 2:
                    ocp[t - 2].wait()
                _reduce_chunk(ib[t % 2], tab_v, ob[t % 2], ch, fan, True,
                              True)
                ocp[t] = pltpu.async_copy(
                    ob[t % 2], out_hbm.at[pl.ds(base + t * ch, ch)],
                    osems[t % 2])
            for t in range(max(nsub - 2, 0), nsub):
                ocp[t].wait()

        @pl.when(w < NUM_WORKERS - 1)
        def _():
            span(nsub_full)

        @pl.when(w == NUM_WORKERS - 1)
        def _():
            span(nsub_tail)

    return k(x_pos, idx)


def _layer1(table, idx):
    fan = F1
    per_w = N2P // NUM_WORKERS
    nsub_full = 4
    ch = per_w // nsub_full
    cw = ch * fan
    slice_w = N1P // LANES
    nsub_tail = (N2 - (NUM_WORKERS - 1) * per_w) // ch

    @functools.partial(
        pl.kernel,
        out_type=jax.ShapeDtypeStruct((N2P,), jnp.float32),
        mesh=_mesh(),
        compiler_params=_compiler_params(),
        scratch_types=[
            pltpu.VMEM_SHARED((N1P,), jnp.float32),
            pltpu.VMEM((cw,), jnp.int32),
            pltpu.VMEM((cw,), jnp.int32),
            pltpu.VMEM((cw,), jnp.float32),
            pltpu.VMEM((cw,), jnp.float32),
            pltpu.VMEM((ch,), jnp.float32),
            pltpu.VMEM((ch,), jnp.float32),
            pltpu.SemaphoreType.DMA,
            pltpu.SemaphoreType.DMA,
            pltpu.SemaphoreType.DMA,
            pltpu.SemaphoreType.DMA,
            pltpu.SemaphoreType.DMA,
            pltpu.SemaphoreType.DMA,
        ],
    )
    def k(tab_hbm, idx_hbm, out_hbm, tab_s, ia, ib_, ga, gb_, oa, ob_,
          isem_a, isem_b, gsem_a, gsem_b, osem_a, osem_b):
        sid = lax.axis_index("s")
        pltpu.sync_copy(tab_hbm.at[pl.ds(sid * slice_w, slice_w)],
                        tab_s.at[pl.ds(sid * slice_w, slice_w)])
        plsc.subcore_barrier()
        w = _wid()
        base = w * per_w
        ib = [ia, ib_]
        gb = [ga, gb_]
        ob = [oa, ob_]
        isems = [isem_a, isem_b]
        gsems = [gsem_a, gsem_b]
        osems = [osem_a, osem_b]

        def span(nsub):
            icp = [None] * nsub
            gcp = [None] * nsub
            ocp = [None] * nsub

            def fire_idx(t):
                return pltpu.async_copy(
                    idx_hbm.at[pl.ds(base * fan + t * cw, cw)],
                    ib[t % 2], isems[t % 2])

            def fire_gather(t):
                src = tab_s if t % 2 == 0 else tab_hbm
                return pltpu.async_copy(src.at[ib[t % 2]], gb[t % 2],
                                        gsems[t % 2])

            icp[0] = fire_idx(0)
            if nsub > 1:
                icp[1] = fire_idx(1)
            icp[0].wait()
            gcp[0] = fire_gather(0)
            for t in range(nsub):
                if t + 1 < nsub:
                    icp[t + 1].wait()
                    gcp[t + 1] = fire_gather(t + 1)
                gcp[t].wait()
                if t + 2 < nsub:
                    icp[t + 2] = fire_idx(t + 2)
                if t >= 2:
                    ocp[t - 2].wait()
                _reduce_positional(gb[t % 2], ob[t % 2], ch, fan, False)
                ocp[t] = pltpu.async_copy(
                    ob[t % 2], out_hbm.at[pl.ds(base + t * ch, ch)],
                    osems[t % 2])
            for t in range(max(nsub - 2, 0), nsub):
                ocp[t].wait()

        @pl.when(w < NUM_WORKERS - 1)
        def _():
            span(nsub_full)

        @pl.when(w == NUM_WORKERS - 1)
        def _():
            span(nsub_tail)

    return k(table, idx)


def _layers23(table, idx2, idx3):
    ch2 = N3P // LANES
    ch3 = N4P // NUM_WORKERS
    slice_w = N2P // LANES

    @functools.partial(
        pl.kernel,
        out_type=jax.ShapeDtypeStruct((N4P,), jnp.float32),
        mesh=_mesh(),
        compiler_params=_compiler_params(),
        scratch_types=[
            pltpu.VMEM_SHARED((N2P,), jnp.float32),
            pltpu.VMEM_SHARED((N3P,), jnp.float32),
            pltpu.VMEM((ch2 * F2,), jnp.int32),
            pltpu.VMEM((ch2 * F2,), jnp.float32),
            pltpu.VMEM((ch2,), jnp.float32),
            pltpu.VMEM((ch3 * F3,), jnp.int32),
            pltpu.VMEM((ch3 * F3,), jnp.float32),
            pltpu.VMEM((ch3,), jnp.float32),
            pltpu.SemaphoreType.DMA,
            pltpu.SemaphoreType.DMA,
        ],
    )
    def k(tab_hbm, idx2_hbm, idx3_hbm, out_hbm, l1_s, l2_s, i2, g2, o2,
          i3, g3, o3, sem_a, sem_b):
        sid = lax.axis_index("s")
        core = lax.axis_index("c")
        base3 = core * (N4P // 2) + sid * ch3
        pltpu.sync_copy(tab_hbm.at[pl.ds(sid * slice_w, slice_w)],
                        l1_s.at[pl.ds(sid * slice_w, slice_w)])
        icp2 = pltpu.async_copy(
            idx2_hbm.at[pl.ds(sid * ch2 * F2, ch2 * F2)], i2, sem_a)
        icp3 = pltpu.async_copy(
            idx3_hbm.at[pl.ds(base3 * F3, ch3 * F3)], i3, sem_b)
        plsc.subcore_barrier()
        icp2.wait()
        pltpu.async_copy(l1_s.at[i2], g2, sem_a).wait()
        _reduce_positional(g2, o2, ch2, F2, True)
        pltpu.sync_copy(o2, l2_s.at[pl.ds(sid * ch2, ch2)])
        plsc.subcore_barrier()
        icp3.wait()
        pltpu.async_copy(l2_s.at[i3], g3, sem_b).wait()
        _reduce_positional(g3, o3, ch3, F3, False)
        pltpu.sync_copy(o3, out_hbm.at[pl.ds(base3, ch3)])

    return k(table, idx2, idx3)


def kernel(x_pos, ix_in0, ix_out0, ix_in1, ix_out1, ix_in2, ix_out2,
           ix_in3, ix_out3):
    del ix_out0, ix_out1, ix_out2, ix_out3

    ix2 = jnp.pad(ix_in2, (0, (N3P - N3) * F2))
    ix3 = jnp.pad(ix_in3, (0, (N4P - N4) * F3))

    l0 = _layer0(x_pos, ix_in0)
    l1 = _layer1(l0, ix_in1)
    l3 = _layers23(l1, ix2, ix3)
    return l3[:N4]
